# baseline (speedup 1.0000x reference)
import jax
import jax.numpy as jnp
from jax import lax
from jax.experimental import pallas as pl
from jax.experimental.pallas import tpu as pltpu

B, C, HW = 8, 64, 1024
K = 1024
N = B * HW
COMMITMENT_COST = 0.25
LOSS_SCALE = (1.0 + COMMITMENT_COST) / (N * C)


def _tournament_argmin(d):
    h = d.shape[0] // 2
    da, db = d[:h], d[h:]
    le = da <= db
    idx = lax.broadcasted_iota(jnp.int32, da.shape, 0) + jnp.where(le, 0, h)
    d = jnp.minimum(da, db)
    while d.shape[0] > 1:
        h = d.shape[0] // 2
        da, db = d[:h], d[h:]
        ia, ib = idx[:h], idx[h:]
        keep_a = (da < db) | ((da == db) & (ia < ib))
        idx = jnp.where(keep_a, ia, ib)
        d = jnp.minimum(da, db)
    return d[0], idx[0]


def _tc_body(x_ref, emb_ref, q_ref, idx_ref, loss_ref):
    b = pl.program_id(0)
    xb = x_ref[0].reshape(C, HW)
    emb = emb_ref[...]
    mm2 = lax.dot_general(emb, xb + xb, (((1,), (0,)), ((), ())),
                          preferred_element_type=jnp.float32)
    x2 = jnp.sum(xb * xb, axis=0)
    e2 = jnp.sum(emb * emb, axis=1)
    d = (x2[None, :] + e2[:, None]) - mm2
    dmin, idx = _tournament_argmin(d)
    idx_ref[pl.ds(b, 1), :] = idx[None, :]
    iota_k = lax.broadcasted_iota(jnp.int32, (K, HW), 0)
    onehot = jnp.where(iota_k == idx[None, :], 1.0, 0.0)
    q = lax.dot_general(emb, onehot, (((0,), (0,)), ((), ())),
                        preferred_element_type=jnp.float32)
    q_ref[0] = q.reshape(C, 32, 32)

    @pl.when(b == 0)
    def _():
        loss_ref[0, 0] = 0.0

    loss_ref[0, 0] += jnp.sum(dmin)

    @pl.when(b == B - 1)
    def _():
        loss_ref[0, 0] *= LOSS_SCALE


def _tc_stage(x3, emb, interpret=False):
    return pl.pallas_call(
        _tc_body,
        grid=(B,),
        in_specs=[
            pl.BlockSpec((1, C, 32, 32), lambda b: (b, 0, 0, 0)),
            pl.BlockSpec((K, C), lambda b: (0, 0)),
        ],
        out_specs=[
            pl.BlockSpec((1, C, 32, 32), lambda b: (b, 0, 0, 0)),
            pl.BlockSpec((B, HW), lambda b: (0, 0)),
            pl.BlockSpec(memory_space=pltpu.SMEM, block_shape=(1, 1),
                         index_map=lambda b: (0, 0)),
        ],
        out_shape=[
            jax.ShapeDtypeStruct((B, C, 32, 32), jnp.float32),
            jax.ShapeDtypeStruct((B, HW), jnp.int32),
            jax.ShapeDtypeStruct((1, 1), jnp.float32),
        ],
        interpret=interpret,
    )(x3, emb)


def kernel(x, emb):
    q4, idx, loss = _tc_stage(x, emb)
    return q4, loss[0, 0], idx


# speedup vs baseline: 1.3176x; 1.3176x over previous
import jax
import jax.numpy as jnp
from jax import lax
from jax.experimental import pallas as pl
from jax.experimental.pallas import tpu as pltpu

B, C, HW = 8, 64, 1024
K = 1024
N = B * HW
COMMITMENT_COST = 0.25
LOSS_SCALE = (1.0 + COMMITMENT_COST) / (N * C)


def _tournament_argmin(d):
    h = d.shape[0] // 2
    da, db = d[:h], d[h:]
    le = da <= db
    idx = lax.broadcasted_iota(jnp.int32, da.shape, 0) + jnp.where(le, 0, h)
    d = jnp.minimum(da, db)
    while d.shape[0] > 1:
        h = d.shape[0] // 2
        da, db = d[:h], d[h:]
        ia, ib = idx[:h], idx[h:]
        keep_a = (da < db) | ((da == db) & (ia < ib))
        idx = jnp.where(keep_a, ia, ib)
        d = jnp.minimum(da, db)
    return d[0], idx[0]


PW = 512
NJ = HW // PW


def _tc_body(x_ref, emb_ref, q_ref, idx_ref, loss_ref):
    b = pl.program_id(0)
    j = pl.program_id(1)
    xb = x_ref[0]
    emb = emb_ref[...]
    mm2 = lax.dot_general(emb, xb + xb, (((1,), (0,)), ((), ())),
                          preferred_element_type=jnp.float32)
    x2 = jnp.sum(xb * xb, axis=0)
    e2 = jnp.sum(emb * emb, axis=1)
    d = (x2[None, :] + e2[:, None]) - mm2
    dmin, idx = _tournament_argmin(d)
    idx_ref[pl.ds(b, 1), pl.ds(j * PW, PW)] = idx[None, :]
    iota_k = lax.broadcasted_iota(jnp.int32, (K, PW), 0)
    onehot = jnp.where(iota_k == idx[None, :], 1.0, 0.0)
    q_ref[0] = lax.dot_general(emb, onehot, (((0,), (0,)), ((), ())),
                               preferred_element_type=jnp.float32)

    @pl.when((b == 0) & (j == 0))
    def _():
        loss_ref[0, 0] = 0.0

    loss_ref[0, 0] += jnp.sum(dmin)

    @pl.when((b == B - 1) & (j == NJ - 1))
    def _():
        loss_ref[0, 0] *= LOSS_SCALE


def _tc_stage(x3, emb, interpret=False):
    return pl.pallas_call(
        _tc_body,
        grid=(B, NJ),
        in_specs=[
            pl.BlockSpec((1, C, PW), lambda b, j: (b, 0, j)),
            pl.BlockSpec((K, C), lambda b, j: (0, 0)),
        ],
        out_specs=[
            pl.BlockSpec((1, C, PW), lambda b, j: (b, 0, j)),
            pl.BlockSpec((B, HW), lambda b, j: (0, 0)),
            pl.BlockSpec(memory_space=pltpu.SMEM, block_shape=(1, 1),
                         index_map=lambda b, j: (0, 0)),
        ],
        out_shape=[
            jax.ShapeDtypeStruct((B, C, HW), jnp.float32),
            jax.ShapeDtypeStruct((B, HW), jnp.int32),
            jax.ShapeDtypeStruct((1, 1), jnp.float32),
        ],
        interpret=interpret,
    )(x3, emb)


def kernel(x, emb):
    x3 = x.reshape(B, C, HW)
    q3, idx, loss = _tc_stage(x3, emb)
    return q3.reshape(B, C, 32, 32), loss[0, 0], idx


# speedup vs baseline: 1.4898x; 1.1307x over previous
import jax
import jax.numpy as jnp
from jax import lax
from jax.experimental import pallas as pl
from jax.experimental.pallas import tpu as pltpu

B, C, HW = 8, 64, 1024
K = 1024
N = B * HW
COMMITMENT_COST = 0.25
LOSS_SCALE = (1.0 + COMMITMENT_COST) / (N * C)


def _tournament_argmin(d):
    h = d.shape[0] // 2
    da, db = d[:h], d[h:]
    le = da <= db
    idx = lax.broadcasted_iota(jnp.int32, da.shape, 0) + jnp.where(le, 0, h)
    d = jnp.minimum(da, db)
    while d.shape[0] > 1:
        h = d.shape[0] // 2
        da, db = d[:h], d[h:]
        ia, ib = idx[:h], idx[h:]
        keep_a = (da < db) | ((da == db) & (ia < ib))
        idx = jnp.where(keep_a, ia, ib)
        d = jnp.minimum(da, db)
    return d[0], idx[0]


def _tc_body(x_ref, emb_ref, q_ref, idx_ref, loss_ref):
    b = pl.program_id(0)
    xb = x_ref[0]
    emb = emb_ref[...]
    mm2 = lax.dot_general(emb, xb + xb, (((1,), (0,)), ((), ())),
                          preferred_element_type=jnp.float32)
    x2 = jnp.sum(xb * xb, axis=0)
    e2 = jnp.sum(emb * emb, axis=1)
    d = (x2[None, :] + e2[:, None]) - mm2
    dmin, idx = _tournament_argmin(d)
    idx_ref[pl.ds(b, 1), :] = idx[None, :]
    iota_k = lax.broadcasted_iota(jnp.int32, (K, HW), 0)
    onehot = jnp.where(iota_k == idx[None, :], 1.0, 0.0)
    q_ref[0] = lax.dot_general(emb, onehot, (((0,), (0,)), ((), ())),
                               preferred_element_type=jnp.float32)

    @pl.when(b == 0)
    def _():
        loss_ref[0, 0] = 0.0

    loss_ref[0, 0] += jnp.sum(dmin)

    @pl.when(b == B - 1)
    def _():
        loss_ref[0, 0] *= LOSS_SCALE


def _tc_stage(x3, emb, interpret=False):
    return pl.pallas_call(
        _tc_body,
        grid=(B,),
        in_specs=[
            pl.BlockSpec((1, C, HW), lambda b: (b, 0, 0)),
            pl.BlockSpec((K, C), lambda b: (0, 0)),
        ],
        out_specs=[
            pl.BlockSpec((1, C, HW), lambda b: (b, 0, 0)),
            pl.BlockSpec((B, HW), lambda b: (0, 0)),
            pl.BlockSpec(memory_space=pltpu.SMEM, block_shape=(1, 1),
                         index_map=lambda b: (0, 0)),
        ],
        out_shape=[
            jax.ShapeDtypeStruct((B, C, HW), jnp.float32),
            jax.ShapeDtypeStruct((B, HW), jnp.int32),
            jax.ShapeDtypeStruct((1, 1), jnp.float32),
        ],
        interpret=interpret,
    )(x3, emb)


def kernel(x, emb):
    x3 = x.reshape(B, C, HW)
    q3, idx, loss = _tc_stage(x3, emb)
    return q3.reshape(B, C, 32, 32), loss[0, 0], idx
